# baseline (speedup 1.0000x reference)
import functools

import jax
import jax.numpy as jnp
import numpy as np
from jax import lax
from jax.experimental import pallas as pl
from jax.experimental.pallas import tpu as pltpu
from jax.experimental.pallas import tpu_sc as plsc

VOCAB = 1000000
D_MODEL = 64
MAX_SEQ = 200
BATCH = 4096
SEQ = 200

NUM_CORES = 2
NUM_SUBCORES = 16
NUM_WORKERS = NUM_CORES * NUM_SUBCORES
LANES = 16

ROWS_PER_WORKER = BATCH // NUM_WORKERS


def _positional_encoding(length, depth):
    half = depth / 2
    positions = np.arange(length)[:, np.newaxis]
    depths = np.arange(half)[np.newaxis, :] / half
    angle_rates = 1 / 10000 ** depths
    angle_rads = positions * angle_rates
    return np.concatenate(
        [np.sin(angle_rads), np.cos(angle_rads)], axis=-1
    ).astype(np.float32)


_PE = _positional_encoding(MAX_SEQ, D_MODEL)
_SCALE = float(np.sqrt(D_MODEL))


def _sc_body(x_hbm, pe_hbm, table_hbm, out_hbm, idx_v, rows_v, pe_v, sem):
    wid = lax.axis_index("s") * NUM_CORES + lax.axis_index("c")
    base = wid * ROWS_PER_WORKER

    pltpu.sync_copy(pe_hbm, pe_v)

    def per_row(b, carry):
        row = base + b
        tok0 = row * SEQ
        pltpu.sync_copy(x_hbm.at[pl.ds(tok0, SEQ)], idx_v)
        pltpu.async_copy(table_hbm.at[idx_v], rows_v, sem).wait()

        def per_pos(s, c2):
            for c in range(D_MODEL // LANES):
                sl = pl.ds(c * LANES, LANES)
                v = rows_v[s, sl]
                p = pe_v[s, sl]
                rows_v[s, sl] = v * _SCALE + p
            return c2

        lax.fori_loop(0, SEQ, per_pos, 0, unroll=2)
        pltpu.sync_copy(rows_v, out_hbm.at[pl.ds(tok0, SEQ)])
        return carry

    lax.fori_loop(0, ROWS_PER_WORKER, per_row, 0)


@jax.jit
def _positional_embedding(x_flat, table, pe):
    mesh = plsc.VectorSubcoreMesh(
        core_axis_name="c", subcore_axis_name="s",
        num_cores=NUM_CORES, num_subcores=NUM_SUBCORES,
    )
    run = pl.kernel(
        _sc_body,
        out_type=jax.ShapeDtypeStruct((BATCH * SEQ, D_MODEL), jnp.float32),
        mesh=mesh,
        scratch_types=[
            pltpu.VMEM((SEQ,), jnp.int32),
            pltpu.VMEM((SEQ, D_MODEL), jnp.float32),
            pltpu.VMEM((SEQ, D_MODEL), jnp.float32),
            pltpu.SemaphoreType.DMA,
        ],
        compiler_params=pltpu.CompilerParams(use_tc_tiling_on_sc=False),
    )
    return run(x_flat, pe, table)


def kernel(x, table):
    x_flat = jnp.reshape(x, (BATCH * SEQ,)).astype(jnp.int32)
    pe = jnp.asarray(_PE)
    out = _positional_embedding(x_flat, table, pe)
    return jnp.reshape(out, (BATCH, SEQ, D_MODEL))
